# baseline (speedup 1.0000x reference)
import functools

import jax
import jax.numpy as jnp
from jax import lax
from jax.experimental import pallas as pl
from jax.experimental.pallas import tpu as pltpu
from jax.experimental.pallas import tpu_sc as plsc

LANES = 16


def _sc_pool(x, table):
    B, SEQ = x.shape
    _, D = table.shape
    NC, NS = 2, 16
    NW = NC * NS
    RPW = B // NW
    C0 = 128
    C1 = SEQ - C0

    mesh = plsc.VectorSubcoreMesh(core_axis_name="c", subcore_axis_name="s")

    @functools.partial(
        pl.kernel,
        out_type=jax.ShapeDtypeStruct((B, D), jnp.float32),
        mesh=mesh,
        scratch_types=[
            pltpu.VMEM((RPW, SEQ), jnp.int32),
            pltpu.VMEM((SEQ, D), jnp.float32),
            pltpu.VMEM((SEQ, D), jnp.float32),
            pltpu.VMEM((RPW, D), jnp.float32),
            pltpu.SemaphoreType.DMA,
            pltpu.SemaphoreType.DMA,
        ],
        compiler_params=pltpu.CompilerParams(
            use_tc_tiling_on_sc=False, needs_layout_passes=False),
    )
    def k(x_hbm, tab_hbm, mean_hbm, xv, rows0, rows1, meanv, sem0, sem1):
        wid = lax.axis_index("s") * NC + lax.axis_index("c")
        base = wid * RPW
        pltpu.sync_copy(x_hbm.at[pl.ds(base, RPW)], xv)

        def issue(i, rows, sem):
            pltpu.async_copy(
                tab_hbm.at[xv.at[i, pl.ds(0, C0)]], rows.at[pl.ds(0, C0)], sem)
            pltpu.async_copy(
                tab_hbm.at[xv.at[i, pl.ds(C0, C1)]], rows.at[pl.ds(C0, C1)], sem)

        def drain(rows, sem):
            pltpu.make_async_copy(tab_hbm.at[pl.ds(0, SEQ)], rows, sem).wait()

        def compute(i, rows):
            H = SEQ // 2

            def body(j, carry):
                a0, a1, b0, b1 = carry
                a0 = a0 + rows[j, pl.ds(0, LANES)]
                a1 = a1 + rows[j, pl.ds(LANES, LANES)]
                b0 = b0 + rows[j + H, pl.ds(0, LANES)]
                b1 = b1 + rows[j + H, pl.ds(LANES, LANES)]
                return a0, a1, b0, b1

            z = jnp.zeros((LANES,), jnp.float32)
            a0, a1, b0, b1 = lax.fori_loop(0, H, body, (z, z, z, z), unroll=10)
            meanv[i, pl.ds(0, LANES)] = a0 + b0
            meanv[i, pl.ds(LANES, LANES)] = a1 + b1

        issue(0, rows0, sem0)

        def body2(t, carry):
            i0 = t * 2
            issue(i0 + 1, rows1, sem1)
            drain(rows0, sem0)
            compute(i0, rows0)

            @pl.when(i0 + 2 < RPW)
            def _():
                issue(i0 + 2, rows0, sem0)

            drain(rows1, sem1)
            compute(i0 + 1, rows1)
            return carry

        lax.fori_loop(0, RPW // 2, body2, 0)
        pltpu.sync_copy(meanv, mean_hbm.at[pl.ds(base, RPW)])

    return k(x, table)


def _tc_relayout(tabT):
    _, V = tabT.shape
    TPB = 128
    BLKC = 512 * TPB
    G = -(-V // BLKC)

    def body(in_ref, o_ref):
        for t in range(TPB):
            p = in_ref[:, t * 512:(t + 1) * 512]
            s = jnp.concatenate(
                [p[:, j * 128:(j + 1) * 128] for j in range(4)], axis=0)
            o_ref[t * 128:(t + 1) * 128, :] = s.T

    return pl.pallas_call(
        body,
        grid=(G,),
        in_specs=[pl.BlockSpec((32, BLKC), lambda i: (0, i))],
        out_specs=pl.BlockSpec((BLKC // 4, 128), lambda i: (i, 0)),
        out_shape=jax.ShapeDtypeStruct((G * BLKC // 4, 128), jnp.float32),
        compiler_params=pltpu.CompilerParams(
            dimension_semantics=("parallel",)),
    )(tabT)


def _tc_sigma(x):
    def body(x_ref, o_ref):
        v = x_ref[...]
        o_ref[...] = (v & -512) + ((v & 127) << 2) + ((v >> 7) & 3)

    return pl.pallas_call(
        body,
        out_shape=jax.ShapeDtypeStruct(x.shape, jnp.int32),
    )(x)


def _mm(summ, x, W, b2):
    B, D = summ.shape
    C, _ = W.shape
    _, L = x.shape
    BLK = 256

    def mmk(s_ref, x_ref, w_ref, b_ref, o_ref):
        cnt = jnp.sum((x_ref[...] != 0).astype(jnp.float32), axis=1,
                      keepdims=True)
        m = s_ref[...] / jnp.maximum(cnt, 1.0)
        o_ref[...] = lax.dot_general(
            m, w_ref[...],
            dimension_numbers=(((1,), (1,)), ((), ())),
            preferred_element_type=jnp.float32,
        ) + b_ref[...]

    return pl.pallas_call(
        mmk,
        grid=(B // BLK,),
        in_specs=[
            pl.BlockSpec((BLK, D), lambda i: (i, 0)),
            pl.BlockSpec((BLK, L), lambda i: (i, 0)),
            pl.BlockSpec((C, D), lambda i: (0, 0)),
            pl.BlockSpec((1, C), lambda i: (0, 0)),
        ],
        out_specs=pl.BlockSpec((BLK, C), lambda i: (i, 0)),
        out_shape=jax.ShapeDtypeStruct((B, C), jnp.float32),
    )(summ, x, W, b2)


@jax.jit
def kernel(x, table, W, b):
    x = x.astype(jnp.int32)
    _, D = table.shape
    packed = _tc_relayout(table.T)
    tab_lin = packed.reshape(packed.shape[0] * 4, D)
    summ = _sc_pool(_tc_sigma(x), tab_lin)
    return _mm(summ, x, W, b.reshape(1, -1))


# speedup vs baseline: 1.0237x; 1.0237x over previous
import functools

import jax
import jax.numpy as jnp
from jax import lax
from jax.experimental import pallas as pl
from jax.experimental.pallas import tpu as pltpu
from jax.experimental.pallas import tpu_sc as plsc

LANES = 16


def _sc_pool(xa, xb, table, seq):
    B, _ = xa.shape
    _, D = table.shape
    NC, NS = 2, 16
    NW = NC * NS
    RPW = B // NW
    C0 = 128
    C1 = seq - C0

    mesh = plsc.VectorSubcoreMesh(core_axis_name="c", subcore_axis_name="s")

    @functools.partial(
        pl.kernel,
        out_type=jax.ShapeDtypeStruct((B, D), jnp.float32),
        mesh=mesh,
        scratch_types=[
            pltpu.VMEM((RPW, C0), jnp.int32),
            pltpu.VMEM((RPW, 128), jnp.int32),
            pltpu.VMEM((seq, D), jnp.float32),
            pltpu.VMEM((seq, D), jnp.float32),
            pltpu.VMEM((RPW, D), jnp.float32),
            pltpu.SemaphoreType.DMA,
            pltpu.SemaphoreType.DMA,
        ],
        compiler_params=pltpu.CompilerParams(
            use_tc_tiling_on_sc=False, needs_layout_passes=False),
    )
    def k(xa_hbm, xb_hbm, tab_hbm, mean_hbm, xva, xvb, rows0, rows1, meanv,
          sem0, sem1):
        wid = lax.axis_index("s") * NC + lax.axis_index("c")
        base = wid * RPW
        pltpu.sync_copy(xa_hbm.at[pl.ds(base, RPW)], xva)
        pltpu.sync_copy(xb_hbm.at[pl.ds(base, RPW)], xvb)

        def issue(i, rows, sem):
            pltpu.async_copy(
                tab_hbm.at[xva.at[i, pl.ds(0, C0)]], rows.at[pl.ds(0, C0)], sem)
            pltpu.async_copy(
                tab_hbm.at[xvb.at[i, pl.ds(0, C1)]], rows.at[pl.ds(C0, C1)], sem)

        def drain(rows, sem):
            pltpu.make_async_copy(tab_hbm.at[pl.ds(0, seq)], rows, sem).wait()

        def compute(i, rows):
            H = seq // 2

            def body(j, carry):
                a0, a1, b0, b1 = carry
                a0 = a0 + rows[j, pl.ds(0, LANES)]
                a1 = a1 + rows[j, pl.ds(LANES, LANES)]
                b0 = b0 + rows[j + H, pl.ds(0, LANES)]
                b1 = b1 + rows[j + H, pl.ds(LANES, LANES)]
                return a0, a1, b0, b1

            z = jnp.zeros((LANES,), jnp.float32)
            a0, a1, b0, b1 = lax.fori_loop(0, H, body, (z, z, z, z), unroll=10)
            a0 = a0 + b0
            a1 = a1 + b1

            cnt = jnp.zeros((LANES,), jnp.int32)
            for kk in range(C0 // LANES):
                chunk = xva[i, pl.ds(kk * LANES, LANES)]
                cnt = cnt + plsc.all_reduce_population_count(chunk != 0)
            for kk in range(C1 // LANES):
                chunk = xvb[i, pl.ds(kk * LANES, LANES)]
                cnt = cnt + plsc.all_reduce_population_count(chunk != 0)
            rem = C1 - (C1 // LANES) * LANES
            if rem:
                lane = lax.iota(jnp.int32, LANES)
                last = xvb[i, pl.ds(C1 - LANES, LANES)]
                cnt = cnt + plsc.all_reduce_population_count(
                    (lane >= LANES - rem) & (last != 0))
            denom = jnp.maximum(cnt.astype(jnp.float32),
                                jnp.ones((LANES,), jnp.float32))
            meanv[i, pl.ds(0, LANES)] = a0 / denom
            meanv[i, pl.ds(LANES, LANES)] = a1 / denom

        issue(0, rows0, sem0)

        def body2(t, carry):
            i0 = t * 2
            issue(i0 + 1, rows1, sem1)
            drain(rows0, sem0)
            compute(i0, rows0)

            @pl.when(i0 + 2 < RPW)
            def _():
                issue(i0 + 2, rows0, sem0)

            drain(rows1, sem1)
            compute(i0 + 1, rows1)
            return carry

        lax.fori_loop(0, RPW // 2, body2, 0)
        pltpu.sync_copy(meanv, mean_hbm.at[pl.ds(base, RPW)])

    return k(xa, xb, table)


def _tc_relayout(tabT):
    _, V = tabT.shape
    TPB = 128
    BLKC = 512 * TPB
    G = -(-V // BLKC)

    def body(in_ref, o_ref):
        for t in range(TPB):
            p = in_ref[:, t * 512:(t + 1) * 512]
            s = jnp.concatenate(
                [p[:, j * 128:(j + 1) * 128] for j in range(4)], axis=0)
            o_ref[t * 128:(t + 1) * 128, :] = s.T

    return pl.pallas_call(
        body,
        grid=(G,),
        in_specs=[pl.BlockSpec((32, BLKC), lambda i: (0, i))],
        out_specs=pl.BlockSpec((BLKC // 4, 128), lambda i: (i, 0)),
        out_shape=jax.ShapeDtypeStruct((G * BLKC // 4, 128), jnp.float32),
        compiler_params=pltpu.CompilerParams(
            dimension_semantics=("parallel",)),
    )(tabT)


def _tc_pack(x):
    B, L = x.shape
    R = 512

    def body(x_ref, oa_ref, ob_ref):
        v = x_ref[...]
        v = (v & -512) + ((v & 127) << 2) + ((v >> 7) & 3)
        oa_ref[...] = v[:, :128]
        ob_ref[...] = v[:, 128:]

    return pl.pallas_call(
        body,
        grid=(B // R,),
        in_specs=[pl.BlockSpec((R, 256), lambda i: (i, 0))],
        out_specs=[pl.BlockSpec((R, 128), lambda i: (i, 0)),
                   pl.BlockSpec((R, 128), lambda i: (i, 0))],
        out_shape=[jax.ShapeDtypeStruct((B, 128), jnp.int32),
                   jax.ShapeDtypeStruct((B, 128), jnp.int32)],
    )(x)


def _mm(mean, W, b2):
    B, D = mean.shape
    C, _ = W.shape
    BLK = 256

    def mmk(m_ref, w_ref, b_ref, o_ref):
        o_ref[...] = lax.dot_general(
            m_ref[...], w_ref[...],
            dimension_numbers=(((1,), (1,)), ((), ())),
            preferred_element_type=jnp.float32,
        ) + b_ref[...]

    return pl.pallas_call(
        mmk,
        grid=(B // BLK,),
        in_specs=[
            pl.BlockSpec((BLK, D), lambda i: (i, 0)),
            pl.BlockSpec((C, D), lambda i: (0, 0)),
            pl.BlockSpec((1, C), lambda i: (0, 0)),
        ],
        out_specs=pl.BlockSpec((BLK, C), lambda i: (i, 0)),
        out_shape=jax.ShapeDtypeStruct((B, C), jnp.float32),
    )(mean, W, b2)


@jax.jit
def kernel(x, table, W, b):
    x = x.astype(jnp.int32)
    _, D = table.shape
    packed = _tc_relayout(table.T)
    tab_lin = packed.reshape(packed.shape[0] * 4, D)
    xa, xb = _tc_pack(x)
    mean = _sc_pool(xa, xb, tab_lin, x.shape[1])
    return _mm(mean, W, b.reshape(1, -1))
